# Initial kernel scaffold; baseline (speedup 1.0000x reference)
import jax
import jax.numpy as jnp
from jax.experimental import pallas as pl


def kernel(x, edge_attr, W1, W2, W3, b3, Wm1, bm1, Wm2, bm2, edge_index, revedge_index, batch, num_nodes):
    raise NotImplementedError("write your pallas kernel here")


# speedup vs baseline: 2.3701x; 2.3701x over previous
import functools
import jax
import jax.numpy as jnp
from jax import lax
from jax.experimental import pallas as pl
from jax.experimental.pallas import tpu as pltpu
from jax.experimental.pallas import tpu_sc as plsc

NC = 2
NS = 16
NW = NC * NS

_E = 640000
_N = 10000
_H = 64
_DEPTH = 3
_NUM_GRAPHS = 64

_SC_PARAMS = pltpu.CompilerParams(use_tc_tiling_on_sc=False)


def _make_sc_gather(T, E, C):
    RW = E // NW
    assert RW % C == 0
    mesh = plsc.VectorSubcoreMesh(core_axis_name="c", subcore_axis_name="s",
                                  num_cores=NC, num_subcores=NS)

    @functools.partial(
        pl.kernel, mesh=mesh,
        out_type=jax.ShapeDtypeStruct((E, _H), jnp.float32),
        compiler_params=_SC_PARAMS,
        scratch_types=[
            pltpu.VMEM((C,), jnp.int32),
            pltpu.VMEM((C, _H), jnp.float32),
            pltpu.SemaphoreType.DMA,
        ],
    )
    def k(table_hbm, idx_hbm, out_hbm, idx_v, rows_v, sem):
        wid = lax.axis_index("s") * NC + lax.axis_index("c")
        base = wid * RW

        def step(i, carry):
            off = base + i * C
            pltpu.sync_copy(idx_hbm.at[pl.ds(off, C)], idx_v)
            pltpu.async_copy(table_hbm.at[idx_v], rows_v, sem).wait()
            pltpu.sync_copy(rows_v, out_hbm.at[pl.ds(off, C)])
            return carry

        lax.fori_loop(0, RW // C, step, 0)

    return k


def _make_sc_segsum(E, N, C):
    RW = E // NW
    assert RW % C == 0
    NPT = N // NS
    ZR = 125
    assert NPT % ZR == 0
    mesh = plsc.VectorSubcoreMesh(core_axis_name="c", subcore_axis_name="s",
                                  num_cores=NC, num_subcores=NS)

    @functools.partial(
        pl.kernel, mesh=mesh,
        out_type=jax.ShapeDtypeStruct((NC, N, _H), jnp.float32),
        compiler_params=_SC_PARAMS,
        scratch_types=[
            pltpu.VMEM((C,), jnp.int32),
            pltpu.VMEM((C, _H), jnp.float32),
            pltpu.VMEM((ZR, _H), jnp.float32),
            pltpu.VMEM_SHARED((N, _H), jnp.float32),
            pltpu.SemaphoreType.DMA,
        ],
    )
    def k(h_hbm, dst_hbm, out_hbm, idx_v, rows_v, zbuf, shared, sem):
        c = lax.axis_index("c")
        s = lax.axis_index("s")
        wid = s * NC + c
        base = wid * RW

        def zrow(i, carry):
            def zvec(j, carry2):
                zbuf[i, pl.ds(j * 16, 16)] = jnp.zeros((16,), jnp.float32)
                return carry2
            return lax.fori_loop(0, _H // 16, zvec, carry)
        lax.fori_loop(0, ZR, zrow, 0)

        def zcopy(kk, carry):
            pltpu.sync_copy(zbuf, shared.at[pl.ds(s * NPT + kk * ZR, ZR)])
            return carry
        lax.fori_loop(0, NPT // ZR, zcopy, 0)
        plsc.subcore_barrier()

        def step(i, carry):
            off = base + i * C
            pltpu.sync_copy(dst_hbm.at[pl.ds(off, C)], idx_v)
            pltpu.sync_copy(h_hbm.at[pl.ds(off, C)], rows_v)
            pltpu.sync_copy(rows_v, shared.at[idx_v], add=True)
            return carry
        lax.fori_loop(0, RW // C, step, 0)
        plsc.subcore_barrier()

        pltpu.sync_copy(shared.at[pl.ds(s * NPT, NPT)],
                        out_hbm.at[c].at[pl.ds(s * NPT, NPT)])

    return k


def _mm_kernel(a_ref, w_ref, o_ref):
    o_ref[...] = jnp.dot(a_ref[...], w_ref[...],
                         preferred_element_type=jnp.float32)


def _tc_matmul(a, w, block_rows):
    M, K = a.shape
    _, Np = w.shape
    grid = M // block_rows
    return pl.pallas_call(
        _mm_kernel,
        grid=(grid,),
        in_specs=[
            pl.BlockSpec((block_rows, K), lambda i: (i, 0)),
            pl.BlockSpec((K, Np), lambda i: (0, 0)),
        ],
        out_specs=pl.BlockSpec((block_rows, Np), lambda i: (i, 0)),
        out_shape=jax.ShapeDtypeStruct((M, Np), jnp.float32),
    )(a, w)


def _h0_kernel(g_ref, ea_ref, w_ref, o_ref):
    o_ref[...] = jax.nn.relu(
        g_ref[...] + jnp.dot(ea_ref[...], w_ref[...],
                             preferred_element_type=jnp.float32))


def _tc_h0(g, edge_attr, w1e, block_rows):
    E = g.shape[0]
    grid = E // block_rows
    return pl.pallas_call(
        _h0_kernel,
        grid=(grid,),
        in_specs=[
            pl.BlockSpec((block_rows, _H), lambda i: (i, 0)),
            pl.BlockSpec((block_rows, edge_attr.shape[1]), lambda i: (i, 0)),
            pl.BlockSpec((edge_attr.shape[1], _H), lambda i: (0, 0)),
        ],
        out_specs=pl.BlockSpec((block_rows, _H), lambda i: (i, 0)),
        out_shape=jax.ShapeDtypeStruct((E, _H), jnp.float32),
    )(g, edge_attr, w1e)


def _combine_mm_kernel(h0_ref, g1_ref, g2_ref, w_ref, h_ref, hw_ref):
    h = jax.nn.relu(h0_ref[...] + g1_ref[...] - g2_ref[...])
    h_ref[...] = h
    hw_ref[...] = jnp.dot(h, w_ref[...], preferred_element_type=jnp.float32)


def _tc_combine_mm(h0, g1, g2, w2, block_rows):
    E = h0.shape[0]
    grid = E // block_rows
    return pl.pallas_call(
        _combine_mm_kernel,
        grid=(grid,),
        in_specs=[
            pl.BlockSpec((block_rows, _H), lambda i: (i, 0)),
            pl.BlockSpec((block_rows, _H), lambda i: (i, 0)),
            pl.BlockSpec((block_rows, _H), lambda i: (i, 0)),
            pl.BlockSpec((_H, _H), lambda i: (0, 0)),
        ],
        out_specs=[
            pl.BlockSpec((block_rows, _H), lambda i: (i, 0)),
            pl.BlockSpec((block_rows, _H), lambda i: (i, 0)),
        ],
        out_shape=[
            jax.ShapeDtypeStruct((E, _H), jnp.float32),
            jax.ShapeDtypeStruct((E, _H), jnp.float32),
        ],
    )(h0, g1, g2, w2)


def _combine_kernel(h0_ref, g1_ref, g2_ref, h_ref):
    h_ref[...] = jax.nn.relu(h0_ref[...] + g1_ref[...] - g2_ref[...])


def _tc_combine(h0, g1, g2, block_rows):
    E = h0.shape[0]
    grid = E // block_rows
    spec = pl.BlockSpec((block_rows, _H), lambda i: (i, 0))
    return pl.pallas_call(
        _combine_kernel,
        grid=(grid,),
        in_specs=[spec, spec, spec],
        out_specs=spec,
        out_shape=jax.ShapeDtypeStruct((E, _H), jnp.float32),
    )(h0, g1, g2)


def _psum_mm_kernel(p_ref, w_ref, o_ref):
    m = p_ref[0] + p_ref[1]
    o_ref[...] = jnp.dot(m, w_ref[...], preferred_element_type=jnp.float32)


def _tc_psum_matmul(p, w, block_rows):
    N = p.shape[1]
    grid = N // block_rows
    return pl.pallas_call(
        _psum_mm_kernel,
        grid=(grid,),
        in_specs=[
            pl.BlockSpec((2, block_rows, _H), lambda i: (0, i, 0)),
            pl.BlockSpec((_H, w.shape[1]), lambda i: (0, 0)),
        ],
        out_specs=pl.BlockSpec((block_rows, w.shape[1]), lambda i: (i, 0)),
        out_shape=jax.ShapeDtypeStruct((N, w.shape[1]), jnp.float32),
    )(p, w)


def _node_kernel(x_ref, p_ref, w3a_ref, w3b_ref, b3_ref, o_ref):
    v = p_ref[0] + p_ref[1]
    z = (jnp.dot(x_ref[...], w3a_ref[...], preferred_element_type=jnp.float32)
         + jnp.dot(v, w3b_ref[...], preferred_element_type=jnp.float32)
         + b3_ref[...])
    o_ref[...] = jax.nn.relu(z)


def _tc_node(x, p, w3a, w3b, b3, block_rows):
    N, K = x.shape
    grid = N // block_rows
    return pl.pallas_call(
        _node_kernel,
        grid=(grid,),
        in_specs=[
            pl.BlockSpec((block_rows, K), lambda i: (i, 0)),
            pl.BlockSpec((2, block_rows, _H), lambda i: (0, i, 0)),
            pl.BlockSpec((K, _H), lambda i: (0, 0)),
            pl.BlockSpec((_H, _H), lambda i: (0, 0)),
            pl.BlockSpec((1, _H), lambda i: (0, 0)),
        ],
        out_specs=pl.BlockSpec((block_rows, _H), lambda i: (i, 0)),
        out_shape=jax.ShapeDtypeStruct((N, _H), jnp.float32),
    )(x, p, w3a, w3b, b3)


def _tail_kernel(na_ref, batch_ref, wm1_ref, bm1_ref, wm2_ref, bm2_ref, o_ref):
    b = batch_ref[...]
    gids = lax.broadcasted_iota(jnp.int32, (_NUM_GRAPHS, b.shape[1]), 0)
    oh = (gids == b).astype(jnp.float32)
    sums = jnp.dot(oh, na_ref[...], preferred_element_type=jnp.float32)
    counts = jnp.sum(oh, axis=1, keepdims=True)
    pooled = sums / jnp.maximum(counts, 1.0)
    z1 = jax.nn.relu(
        jnp.dot(pooled, wm1_ref[...], preferred_element_type=jnp.float32)
        + bm1_ref[...])
    out = (jnp.dot(z1, wm2_ref[...], preferred_element_type=jnp.float32)
           + bm2_ref[...])
    o_ref[...] = out * (1.0 / jnp.sqrt(jnp.float32(1.0 + 1e-5)))


def _tc_tail(node_attr, batch2d, Wm1, bm1, Wm2, bm2):
    return pl.pallas_call(
        _tail_kernel,
        out_shape=jax.ShapeDtypeStruct((_NUM_GRAPHS, 1), jnp.float32),
    )(node_attr, batch2d, Wm1, bm1.reshape(1, _H), Wm2, bm2.reshape(1, 1))


_sc_gather_nodes = None
_sc_gather_edges = None
_sc_segsum = None


def _get_sc_kernels():
    global _sc_gather_nodes, _sc_gather_edges, _sc_segsum
    if _sc_gather_nodes is None:
        _sc_gather_nodes = _make_sc_gather(_N, _E, 1000)
        _sc_gather_edges = _make_sc_gather(_E, _E, 1000)
        _sc_segsum = _make_sc_segsum(_E, _N, 1000)
    return _sc_gather_nodes, _sc_gather_edges, _sc_segsum


def kernel(x, edge_attr, W1, W2, W3, b3, Wm1, bm1, Wm2, bm2,
           edge_index, revedge_index, batch, num_nodes):
    gather_n, gather_e, segsum = _get_sc_kernels()
    src = edge_index[0]
    dst = edge_index[1]
    B_E = 5000
    B_N = 2000

    W1x = W1[:x.shape[1]]
    W1e = W1[x.shape[1]:]
    W3a = W3[:x.shape[1]]
    W3b = W3[x.shape[1]:]

    xW1 = _tc_matmul(x, W1x, B_N)
    g0 = gather_n(xW1, src)
    h0 = _tc_h0(g0, edge_attr, W1e, B_E)

    h = h0
    for it in range(_DEPTH - 1):
        p = segsum(h, dst)
        mW2 = _tc_psum_matmul(p, W2, B_N)
        hW2 = _tc_matmul(h, W2, B_E)
        g1 = gather_n(mW2, src)
        g2 = gather_e(hW2, revedge_index)
        h = _tc_combine(h0, g1, g2, B_E)

    p = segsum(h, dst)
    node_attr = _tc_node(x, p, W3a, W3b, b3.reshape(1, _H), B_N)
    out = _tc_tail(node_attr, batch.reshape(1, -1), Wm1, bm1, Wm2, bm2)
    return out


# speedup vs baseline: 5.6561x; 2.3865x over previous
import functools
import jax
import jax.numpy as jnp
from jax import lax
from jax.experimental import pallas as pl
from jax.experimental.pallas import tpu as pltpu
from jax.experimental.pallas import tpu_sc as plsc

NC = 2
NS = 16
NW = NC * NS

_E = 640000
_E2 = _E // 2
_N = 10000
_H = 64
_DEPTH = 3
_NUM_GRAPHS = 64

_SC_PARAMS = pltpu.CompilerParams(use_tc_tiling_on_sc=False)


def _make_sc_gather(T, E, C):
    RW = E // NW
    assert RW % C == 0
    mesh = plsc.VectorSubcoreMesh(core_axis_name="c", subcore_axis_name="s",
                                  num_cores=NC, num_subcores=NS)

    @functools.partial(
        pl.kernel, mesh=mesh,
        out_type=jax.ShapeDtypeStruct((E, _H), jnp.float32),
        compiler_params=_SC_PARAMS,
        scratch_types=[
            pltpu.VMEM((C,), jnp.int32),
            pltpu.VMEM((C, _H), jnp.float32),
            pltpu.SemaphoreType.DMA,
        ],
    )
    def k(table_hbm, idx_hbm, out_hbm, idx_v, rows_v, sem):
        wid = lax.axis_index("s") * NC + lax.axis_index("c")
        base = wid * RW

        def step(i, carry):
            off = base + i * C
            pltpu.sync_copy(idx_hbm.at[pl.ds(off, C)], idx_v)
            pltpu.async_copy(table_hbm.at[idx_v], rows_v, sem).wait()
            pltpu.sync_copy(rows_v, out_hbm.at[pl.ds(off, C)])
            return carry

        lax.fori_loop(0, RW // C, step, 0)

    return k


def _make_sc_segsum(E, N, C):
    RW = E // NW
    assert RW % C == 0
    NPT = N // NS
    ZR = 125
    assert NPT % ZR == 0
    mesh = plsc.VectorSubcoreMesh(core_axis_name="c", subcore_axis_name="s",
                                  num_cores=NC, num_subcores=NS)

    @functools.partial(
        pl.kernel, mesh=mesh,
        out_type=jax.ShapeDtypeStruct((NC, N, _H), jnp.float32),
        compiler_params=_SC_PARAMS,
        scratch_types=[
            pltpu.VMEM((C,), jnp.int32),
            pltpu.VMEM((C, _H), jnp.float32),
            pltpu.VMEM((ZR, _H), jnp.float32),
            pltpu.VMEM_SHARED((N, _H), jnp.float32),
            pltpu.SemaphoreType.DMA,
        ],
    )
    def k(h_hbm, dst_hbm, out_hbm, idx_v, rows_v, zbuf, shared, sem):
        hv = h_hbm
        c = lax.axis_index("c")
        s = lax.axis_index("s")
        wid = s * NC + c
        base = wid * RW

        def zrow(i, carry):
            def zvec(j, carry2):
                zbuf[i, pl.ds(j * 16, 16)] = jnp.zeros((16,), jnp.float32)
                return carry2
            return lax.fori_loop(0, _H // 16, zvec, carry)
        lax.fori_loop(0, ZR, zrow, 0)

        def zcopy(kk, carry):
            pltpu.sync_copy(zbuf, shared.at[pl.ds(s * NPT + kk * ZR, ZR)])
            return carry
        lax.fori_loop(0, NPT // ZR, zcopy, 0)
        plsc.subcore_barrier()

        def step(i, carry):
            off = base + i * C
            pltpu.sync_copy(dst_hbm.at[pl.ds(off, C)], idx_v)
            pltpu.sync_copy(hv.at[pl.ds(off, C)], rows_v)
            pltpu.sync_copy(rows_v, shared.at[idx_v], add=True)
            return carry
        lax.fori_loop(0, RW // C, step, 0)
        plsc.subcore_barrier()

        pltpu.sync_copy(shared.at[pl.ds(s * NPT, NPT)],
                        out_hbm.at[c].at[pl.ds(s * NPT, NPT)])

    return k


def _mm_kernel(a_ref, w_ref, o_ref):
    o_ref[...] = jnp.dot(a_ref[...], w_ref[...],
                         preferred_element_type=jnp.float32)


def _tc_matmul(a, w, block_rows):
    M, K = a.shape
    _, Np = w.shape
    grid = M // block_rows
    return pl.pallas_call(
        _mm_kernel,
        grid=(grid,),
        in_specs=[
            pl.BlockSpec((block_rows, K), lambda i: (i, 0)),
            pl.BlockSpec((K, Np), lambda i: (0, 0)),
        ],
        out_specs=pl.BlockSpec((block_rows, Np), lambda i: (i, 0)),
        out_shape=jax.ShapeDtypeStruct((M, Np), jnp.float32),
    )(a, w)


def _h0_kernel(g_ref, ea2_ref, wbig_ref, h0_ref):
    ewp = jnp.dot(ea2_ref[...], wbig_ref[...],
                  preferred_element_type=jnp.float32)
    h0_ref[...] = jax.nn.relu(g_ref[...] + ewp)


def _tc_h0(g0p, ea2, wbig, block_rows):
    grid = _E2 // block_rows
    pspec = pl.BlockSpec((block_rows, 128), lambda i: (i, 0))
    return pl.pallas_call(
        _h0_kernel,
        grid=(grid,),
        in_specs=[
            pspec,
            pl.BlockSpec((block_rows, ea2.shape[1]), lambda i: (i, 0)),
            pl.BlockSpec((ea2.shape[1], 128), lambda i: (0, 0)),
        ],
        out_specs=pspec,
        out_shape=jax.ShapeDtypeStruct((_E2, 128), jnp.float32),
    )(g0p, ea2, wbig)


def _combine_kernel(h0_ref, g1_ref, g2_ref, w_ref, h_ref):
    d = g1_ref[...] - g2_ref[...]
    w2 = w_ref[...]
    ml = jnp.dot(d[:, :_H], w2, preferred_element_type=jnp.float32)
    mr = jnp.dot(d[:, _H:], w2, preferred_element_type=jnp.float32)
    h_ref[:, :_H] = jax.nn.relu(h0_ref[:, :_H] + ml)
    h_ref[:, _H:] = jax.nn.relu(h0_ref[:, _H:] + mr)


def _tc_combine(h0p, g1p, g2p, w2, block_rows):
    grid = _E2 // block_rows
    pspec = pl.BlockSpec((block_rows, 128), lambda i: (i, 0))
    return pl.pallas_call(
        _combine_kernel,
        grid=(grid,),
        in_specs=[pspec, pspec, pspec,
                  pl.BlockSpec((_H, _H), lambda i: (0, 0))],
        out_specs=pspec,
        out_shape=jax.ShapeDtypeStruct((_E2, 128), jnp.float32),
    )(h0p, g1p, g2p, w2)


def _psum_kernel(p_ref, o_ref):
    o_ref[...] = p_ref[0] + p_ref[1]


def _tc_psum(p, block_rows):
    N = p.shape[1]
    grid = N // block_rows
    return pl.pallas_call(
        _psum_kernel,
        grid=(grid,),
        in_specs=[pl.BlockSpec((2, block_rows, _H), lambda i: (0, i, 0))],
        out_specs=pl.BlockSpec((block_rows, _H), lambda i: (i, 0)),
        out_shape=jax.ShapeDtypeStruct((N, _H), jnp.float32),
    )(p)


def _node_kernel(x_ref, p_ref, w3a_ref, w3b_ref, b3_ref, o_ref):
    v = p_ref[0] + p_ref[1]
    z = (jnp.dot(x_ref[...], w3a_ref[...], preferred_element_type=jnp.float32)
         + jnp.dot(v, w3b_ref[...], preferred_element_type=jnp.float32)
         + b3_ref[...])
    o_ref[...] = jax.nn.relu(z)


def _tc_node(x, p, w3a, w3b, b3, block_rows):
    N, K = x.shape
    grid = N // block_rows
    return pl.pallas_call(
        _node_kernel,
        grid=(grid,),
        in_specs=[
            pl.BlockSpec((block_rows, K), lambda i: (i, 0)),
            pl.BlockSpec((2, block_rows, _H), lambda i: (0, i, 0)),
            pl.BlockSpec((K, _H), lambda i: (0, 0)),
            pl.BlockSpec((_H, _H), lambda i: (0, 0)),
            pl.BlockSpec((1, _H), lambda i: (0, 0)),
        ],
        out_specs=pl.BlockSpec((block_rows, _H), lambda i: (i, 0)),
        out_shape=jax.ShapeDtypeStruct((N, _H), jnp.float32),
    )(x, p, w3a, w3b, b3)


def _tail_kernel(na_ref, batch_ref, wm1_ref, bm1_ref, wm2_ref, bm2_ref, o_ref):
    b = batch_ref[...]
    gids = lax.broadcasted_iota(jnp.int32, (_NUM_GRAPHS, b.shape[1]), 0)
    oh = (gids == b).astype(jnp.float32)
    sums = jnp.dot(oh, na_ref[...], preferred_element_type=jnp.float32,
                   precision=lax.Precision.HIGHEST)
    counts = jnp.sum(oh, axis=1, keepdims=True)
    pooled = sums / jnp.maximum(counts, 1.0)
    z1 = jax.nn.relu(
        jnp.dot(pooled, wm1_ref[...], preferred_element_type=jnp.float32)
        + bm1_ref[...])
    out = (jnp.dot(z1, wm2_ref[...], preferred_element_type=jnp.float32)
           + bm2_ref[...])
    o_ref[...] = out * (1.0 / jnp.sqrt(jnp.float32(1.0 + 1e-5)))


def _tc_tail(node_attr, batch2d, Wm1, bm1, Wm2, bm2):
    return pl.pallas_call(
        _tail_kernel,
        out_shape=jax.ShapeDtypeStruct((_NUM_GRAPHS, 1), jnp.float32),
    )(node_attr, batch2d, Wm1, bm1.reshape(1, _H), Wm2, bm2.reshape(1, 1))


_sc_cache = {}


def _get_sc_kernels():
    if not _sc_cache:
        _sc_cache["gn"] = _make_sc_gather(_N, _E, 1000)
        _sc_cache["ge"] = _make_sc_gather(_E, _E, 1000)
        _sc_cache["ss"] = _make_sc_segsum(_E, _N, 1000)
    return _sc_cache["gn"], _sc_cache["ge"], _sc_cache["ss"]


def kernel(x, edge_attr, W1, W2, W3, b3, Wm1, bm1, Wm2, bm2,
           edge_index, revedge_index, batch, num_nodes):
    gather_n, gather_e, segsum = _get_sc_kernels()
    src = edge_index[0]
    dst = edge_index[1]
    B_E = 2000
    B_N = 2000

    W1x = W1[:x.shape[1]]
    W1e = W1[x.shape[1]:]
    W3a = W3[:x.shape[1]]
    W3b = W3[x.shape[1]:]

    def pack(a):
        return jnp.reshape(a, (_E2, 128))

    def unpack(a):
        return jnp.reshape(a, (_E, _H))

    EI = edge_attr.shape[1]
    ea2 = jnp.reshape(edge_attr, (_E2, 2 * EI))
    zpad = jnp.zeros((EI, _H), jnp.float32)
    wbig = jnp.concatenate([
        jnp.concatenate([W1e, zpad], axis=1),
        jnp.concatenate([zpad, W1e], axis=1),
    ], axis=0)
    xW1 = _tc_matmul(x, W1x, B_N)
    g0p = pack(gather_n(xW1, src))
    h0p = _tc_h0(g0p, ea2, wbig, B_E)

    hp = h0p
    for it in range(_DEPTH - 1):
        p = segsum(unpack(hp), dst)
        m = _tc_psum(p, B_N)
        g1p = pack(gather_n(m, src))
        g2p = pack(gather_e(unpack(hp), revedge_index))
        hp = _tc_combine(h0p, g1p, g2p, W2, B_E)

    p = segsum(unpack(hp), dst)
    node_attr = _tc_node(x, p, W3a, W3b, b3.reshape(1, _H), B_N)
    out = _tc_tail(node_attr, batch.reshape(1, -1), Wm1, bm1, Wm2, bm2)
    return out


# speedup vs baseline: 5.9107x; 1.0450x over previous
import functools
import jax
import jax.numpy as jnp
from jax import lax
from jax.experimental import pallas as pl
from jax.experimental.pallas import tpu as pltpu
from jax.experimental.pallas import tpu_sc as plsc

NC = 2
NS = 16
NW = NC * NS

_E = 640000
_E2 = _E // 2
_N = 10000
_H = 64
_DEPTH = 3
_NUM_GRAPHS = 64

_SC_PARAMS = pltpu.CompilerParams(use_tc_tiling_on_sc=False)


def _make_sc_gather(T, E, C):
    RW = E // NW
    n = RW // C
    assert RW % C == 0 and n % 2 == 1
    npairs = (n - 1) // 2
    mesh = plsc.VectorSubcoreMesh(core_axis_name="c", subcore_axis_name="s",
                                  num_cores=NC, num_subcores=NS)

    @functools.partial(
        pl.kernel, mesh=mesh,
        out_type=jax.ShapeDtypeStruct((E, _H), jnp.float32),
        compiler_params=_SC_PARAMS,
        scratch_types=[
            pltpu.VMEM((RW,), jnp.int32),
            pltpu.VMEM((C, _H), jnp.float32),
            pltpu.VMEM((C, _H), jnp.float32),
            pltpu.SemaphoreType.DMA,
            pltpu.SemaphoreType.DMA,
        ],
    )
    def k(table_hbm, idx_hbm, out_hbm, idx_all, rb0, rb1, gsem, wsem):
        wid = lax.axis_index("s") * NC + lax.axis_index("c")
        base = wid * RW
        pltpu.sync_copy(idx_hbm.at[pl.ds(base, RW)], idx_all)

        def idxs(g):
            return idx_all.at[pl.ds(g * C, C)]

        def outs(g):
            return out_hbm.at[pl.ds(base + g * C, C)]

        def drain(rb, sem):
            pltpu.make_async_copy(out_hbm.at[pl.ds(base, C)], rb, sem).wait()

        pltpu.async_copy(table_hbm.at[idxs(0)], rb0, gsem)

        def body(t, first):
            g = 1 + 2 * t
            drain(rb0, gsem)
            if not first:
                drain(rb1, wsem)
            pltpu.async_copy(table_hbm.at[idxs(g)], rb1, gsem)
            pltpu.async_copy(rb0, outs(g - 1), wsem)
            drain(rb1, gsem)
            drain(rb0, wsem)
            pltpu.async_copy(table_hbm.at[idxs(g + 1)], rb0, gsem)
            pltpu.async_copy(rb1, outs(g), wsem)

        body(0, True)

        def fbody(t, carry):
            body(t, False)
            return carry
        lax.fori_loop(1, npairs, fbody, 0)

        drain(rb0, gsem)
        drain(rb1, wsem)
        pltpu.async_copy(rb0, outs(n - 1), wsem)
        drain(rb0, wsem)

    return k


def _make_sc_segsum(E, N, C):
    RW = E // NW
    n = RW // C
    assert RW % C == 0 and n % 2 == 0
    NPT = N // NS
    assert NPT > C and NPT < 2 * C
    mesh = plsc.VectorSubcoreMesh(core_axis_name="c", subcore_axis_name="s",
                                  num_cores=NC, num_subcores=NS)

    @functools.partial(
        pl.kernel, mesh=mesh,
        out_type=jax.ShapeDtypeStruct((NC, N, _H), jnp.float32),
        compiler_params=_SC_PARAMS,
        scratch_types=[
            pltpu.VMEM((C,), jnp.int32),
            pltpu.VMEM((C,), jnp.int32),
            pltpu.VMEM((C, _H), jnp.float32),
            pltpu.VMEM((C, _H), jnp.float32),
            pltpu.VMEM_SHARED((N, _H), jnp.float32),
            pltpu.SemaphoreType.DMA,
            pltpu.SemaphoreType.DMA,
        ],
    )
    def k(h_hbm, dst_hbm, out_hbm, idx0, idx1, rb0, rb1, shared, rsem, ssem):
        c = lax.axis_index("c")
        s = lax.axis_index("s")
        wid = s * NC + c
        base = wid * RW
        idxb = (idx0, idx1)
        rbb = (rb0, rb1)

        def zrow(i, carry):
            def zvec(j, carry2):
                rb1[i, pl.ds(j * 16, 16)] = jnp.zeros((16,), jnp.float32)
                return carry2
            return lax.fori_loop(0, _H // 16, zvec, carry)
        lax.fori_loop(0, C, zrow, 0)
        pltpu.sync_copy(rb1, shared.at[pl.ds(s * NPT, C)])
        pltpu.sync_copy(rb1.at[pl.ds(0, NPT - C)],
                        shared.at[pl.ds(s * NPT + C, NPT - C)])
        plsc.subcore_barrier()

        def ldidx(g, b):
            pltpu.sync_copy(dst_hbm.at[pl.ds(base + g * C, C)], idxb[b])

        def ldrows(g, b):
            pltpu.async_copy(h_hbm.at[pl.ds(base + g * C, C)], rbb[b], rsem)

        def scat(b):
            pltpu.async_copy(rbb[b], shared.at[idxb[b]], ssem, add=True)

        def drain(sem):
            pltpu.make_async_copy(h_hbm.at[pl.ds(base, C)], rb0, sem).wait()

        ldidx(0, 0)
        ldrows(0, 0)

        def body(t, first, last):
            g = 2 * t
            drain(rsem)
            if not first:
                drain(ssem)
            ldidx(g + 1, 1)
            ldrows(g + 1, 1)
            scat(0)
            drain(rsem)
            drain(ssem)
            if not last:
                ldidx(g + 2, 0)
                ldrows(g + 2, 0)
            scat(1)

        body(0, True, False)

        def fbody(t, carry):
            body(t, False, False)
            return carry
        lax.fori_loop(1, n // 2 - 1, fbody, 0)

        body(n // 2 - 1, False, True)
        drain(ssem)
        plsc.subcore_barrier()

        pltpu.sync_copy(shared.at[pl.ds(s * NPT, NPT)],
                        out_hbm.at[c].at[pl.ds(s * NPT, NPT)])

    return k


def _mm_kernel(a_ref, w_ref, o_ref):
    o_ref[...] = jnp.dot(a_ref[...], w_ref[...],
                         preferred_element_type=jnp.float32)


def _tc_matmul(a, w, block_rows):
    M, K = a.shape
    _, Np = w.shape
    grid = M // block_rows
    return pl.pallas_call(
        _mm_kernel,
        grid=(grid,),
        in_specs=[
            pl.BlockSpec((block_rows, K), lambda i: (i, 0)),
            pl.BlockSpec((K, Np), lambda i: (0, 0)),
        ],
        out_specs=pl.BlockSpec((block_rows, Np), lambda i: (i, 0)),
        out_shape=jax.ShapeDtypeStruct((M, Np), jnp.float32),
    )(a, w)


def _h0_kernel(g_ref, ea2_ref, wbig_ref, h0_ref):
    ewp = jnp.dot(ea2_ref[...], wbig_ref[...],
                  preferred_element_type=jnp.float32)
    h0_ref[...] = jax.nn.relu(g_ref[...] + ewp)


def _tc_h0(g0p, ea2, wbig, block_rows):
    grid = _E2 // block_rows
    pspec = pl.BlockSpec((block_rows, 128), lambda i: (i, 0))
    return pl.pallas_call(
        _h0_kernel,
        grid=(grid,),
        in_specs=[
            pspec,
            pl.BlockSpec((block_rows, ea2.shape[1]), lambda i: (i, 0)),
            pl.BlockSpec((ea2.shape[1], 128), lambda i: (0, 0)),
        ],
        out_specs=pspec,
        out_shape=jax.ShapeDtypeStruct((_E2, 128), jnp.float32),
    )(g0p, ea2, wbig)


def _combine_kernel(h0_ref, g1_ref, g2_ref, w_ref, h_ref):
    d = g1_ref[...] - g2_ref[...]
    w2 = w_ref[...]
    ml = jnp.dot(d[:, :_H], w2, preferred_element_type=jnp.float32)
    mr = jnp.dot(d[:, _H:], w2, preferred_element_type=jnp.float32)
    h_ref[:, :_H] = jax.nn.relu(h0_ref[:, :_H] + ml)
    h_ref[:, _H:] = jax.nn.relu(h0_ref[:, _H:] + mr)


def _tc_combine(h0p, g1p, g2p, w2, block_rows):
    grid = _E2 // block_rows
    pspec = pl.BlockSpec((block_rows, 128), lambda i: (i, 0))
    return pl.pallas_call(
        _combine_kernel,
        grid=(grid,),
        in_specs=[pspec, pspec, pspec,
                  pl.BlockSpec((_H, _H), lambda i: (0, 0))],
        out_specs=pspec,
        out_shape=jax.ShapeDtypeStruct((_E2, 128), jnp.float32),
    )(h0p, g1p, g2p, w2)


def _psum_kernel(p_ref, o_ref):
    o_ref[...] = p_ref[0] + p_ref[1]


def _tc_psum(p, block_rows):
    N = p.shape[1]
    grid = N // block_rows
    return pl.pallas_call(
        _psum_kernel,
        grid=(grid,),
        in_specs=[pl.BlockSpec((2, block_rows, _H), lambda i: (0, i, 0))],
        out_specs=pl.BlockSpec((block_rows, _H), lambda i: (i, 0)),
        out_shape=jax.ShapeDtypeStruct((N, _H), jnp.float32),
    )(p)


def _node_kernel(x_ref, p_ref, w3a_ref, w3b_ref, b3_ref, o_ref):
    v = p_ref[0] + p_ref[1]
    z = (jnp.dot(x_ref[...], w3a_ref[...], preferred_element_type=jnp.float32)
         + jnp.dot(v, w3b_ref[...], preferred_element_type=jnp.float32)
         + b3_ref[...])
    o_ref[...] = jax.nn.relu(z)


def _tc_node(x, p, w3a, w3b, b3, block_rows):
    N, K = x.shape
    grid = N // block_rows
    return pl.pallas_call(
        _node_kernel,
        grid=(grid,),
        in_specs=[
            pl.BlockSpec((block_rows, K), lambda i: (i, 0)),
            pl.BlockSpec((2, block_rows, _H), lambda i: (0, i, 0)),
            pl.BlockSpec((K, _H), lambda i: (0, 0)),
            pl.BlockSpec((_H, _H), lambda i: (0, 0)),
            pl.BlockSpec((1, _H), lambda i: (0, 0)),
        ],
        out_specs=pl.BlockSpec((block_rows, _H), lambda i: (i, 0)),
        out_shape=jax.ShapeDtypeStruct((N, _H), jnp.float32),
    )(x, p, w3a, w3b, b3)


def _tail_kernel(na_ref, batch_ref, wm1_ref, bm1_ref, wm2_ref, bm2_ref, o_ref):
    b = batch_ref[...]
    gids = lax.broadcasted_iota(jnp.int32, (_NUM_GRAPHS, b.shape[1]), 0)
    oh = (gids == b).astype(jnp.float32)
    sums = jnp.dot(oh, na_ref[...], preferred_element_type=jnp.float32,
                   precision=lax.Precision.HIGHEST)
    counts = jnp.sum(oh, axis=1, keepdims=True)
    pooled = sums / jnp.maximum(counts, 1.0)
    z1 = jax.nn.relu(
        jnp.dot(pooled, wm1_ref[...], preferred_element_type=jnp.float32)
        + bm1_ref[...])
    out = (jnp.dot(z1, wm2_ref[...], preferred_element_type=jnp.float32)
           + bm2_ref[...])
    o_ref[...] = out * (1.0 / jnp.sqrt(jnp.float32(1.0 + 1e-5)))


def _tc_tail(node_attr, batch2d, Wm1, bm1, Wm2, bm2):
    return pl.pallas_call(
        _tail_kernel,
        out_shape=jax.ShapeDtypeStruct((_NUM_GRAPHS, 1), jnp.float32),
    )(node_attr, batch2d, Wm1, bm1.reshape(1, _H), Wm2, bm2.reshape(1, 1))


_sc_cache = {}


def _get_sc_kernels():
    if not _sc_cache:
        _sc_cache["gn"] = _make_sc_gather(_N, _E, 800)
        _sc_cache["ge"] = _make_sc_gather(_E, _E, 800)
        _sc_cache["ss"] = _make_sc_segsum(_E, _N, 400)
    return _sc_cache["gn"], _sc_cache["ge"], _sc_cache["ss"]


def kernel(x, edge_attr, W1, W2, W3, b3, Wm1, bm1, Wm2, bm2,
           edge_index, revedge_index, batch, num_nodes):
    gather_n, gather_e, segsum = _get_sc_kernels()
    src = edge_index[0]
    dst = edge_index[1]
    B_E = 2000
    B_N = 2000

    W1x = W1[:x.shape[1]]
    W1e = W1[x.shape[1]:]
    W3a = W3[:x.shape[1]]
    W3b = W3[x.shape[1]:]

    def pack(a):
        return jnp.reshape(a, (_E2, 128))

    def unpack(a):
        return jnp.reshape(a, (_E, _H))

    EI = edge_attr.shape[1]
    ea2 = jnp.reshape(edge_attr, (_E2, 2 * EI))
    zpad = jnp.zeros((EI, _H), jnp.float32)
    wbig = jnp.concatenate([
        jnp.concatenate([W1e, zpad], axis=1),
        jnp.concatenate([zpad, W1e], axis=1),
    ], axis=0)
    xW1 = _tc_matmul(x, W1x, B_N)
    g0p = pack(gather_n(xW1, src))
    h0p = _tc_h0(g0p, ea2, wbig, B_E)

    hp = h0p
    for it in range(_DEPTH - 1):
        p = segsum(unpack(hp), dst)
        m = _tc_psum(p, B_N)
        g1p = pack(gather_n(m, src))
        g2p = pack(gather_e(unpack(hp), revedge_index))
        hp = _tc_combine(h0p, g1p, g2p, W2, B_E)

    p = segsum(unpack(hp), dst)
    node_attr = _tc_node(x, p, W3a, W3b, b3.reshape(1, _H), B_N)
    out = _tc_tail(node_attr, batch.reshape(1, -1), Wm1, bm1, Wm2, bm2)
    return out


# speedup vs baseline: 6.0638x; 1.0259x over previous
import functools
import jax
import jax.numpy as jnp
from jax import lax
from jax.experimental import pallas as pl
from jax.experimental.pallas import tpu as pltpu
from jax.experimental.pallas import tpu_sc as plsc

NC = 2
NS = 16
NW = NC * NS

_E = 640000
_E2 = _E // 2
_N = 10000
_H = 64
_DEPTH = 3
_NUM_GRAPHS = 64

_SC_PARAMS = pltpu.CompilerParams(use_tc_tiling_on_sc=False)


def _make_sc_gather(T, E, C):
    RW = E // NW
    E2 = E // 2
    n = RW // C
    assert RW % C == 0 and n % 2 == 1
    npairs = (n - 1) // 2
    mesh = plsc.VectorSubcoreMesh(core_axis_name="c", subcore_axis_name="s",
                                  num_cores=NC, num_subcores=NS)

    @functools.partial(
        pl.kernel, mesh=mesh,
        out_type=jax.ShapeDtypeStruct((E2, 2 * _H), jnp.float32),
        compiler_params=_SC_PARAMS,
        scratch_types=[
            pltpu.VMEM((RW,), jnp.int32),
            pltpu.VMEM((C, _H), jnp.float32),
            pltpu.VMEM((C, _H), jnp.float32),
            pltpu.SemaphoreType.DMA,
            pltpu.SemaphoreType.DMA,
        ],
    )
    def k(table_hbm, idx_hbm, out_hbm, idx_all, rb0, rb1, gsem, wsem):
        wid = lax.axis_index("s") * NC + lax.axis_index("c")
        base = wid * RW
        hf = wid // NS
        prow = base - hf * E2
        pltpu.sync_copy(idx_hbm.at[pl.ds(base, RW)], idx_all)

        def idxs(g):
            return idx_all.at[pl.ds(g * C, C)]

        def outs(g):
            return out_hbm.at[pl.ds(prow + g * C, C), pl.ds(hf * _H, _H)]

        def drain(rb, sem):
            pltpu.make_async_copy(out_hbm.at[pl.ds(0, C), pl.ds(0, _H)],
                                  rb, sem).wait()

        pltpu.async_copy(table_hbm.at[idxs(0)], rb0, gsem)

        def body(t, first):
            g = 1 + 2 * t
            drain(rb0, gsem)
            if not first:
                drain(rb1, wsem)
            pltpu.async_copy(table_hbm.at[idxs(g)], rb1, gsem)
            pltpu.async_copy(rb0, outs(g - 1), wsem)
            drain(rb1, gsem)
            drain(rb0, wsem)
            pltpu.async_copy(table_hbm.at[idxs(g + 1)], rb0, gsem)
            pltpu.async_copy(rb1, outs(g), wsem)

        body(0, True)

        def fbody(t, carry):
            body(t, False)
            return carry
        lax.fori_loop(1, npairs, fbody, 0)

        drain(rb0, gsem)
        drain(rb1, wsem)
        pltpu.async_copy(rb0, outs(n - 1), wsem)
        drain(rb0, wsem)

    return k


def _make_sc_segsum(E, N, C):
    RW = E // NW
    n = RW // C
    assert RW % C == 0 and n % 2 == 0
    NPT = N // NS
    assert NPT > C and NPT < 2 * C
    mesh = plsc.VectorSubcoreMesh(core_axis_name="c", subcore_axis_name="s",
                                  num_cores=NC, num_subcores=NS)

    @functools.partial(
        pl.kernel, mesh=mesh,
        out_type=jax.ShapeDtypeStruct((NC, N, _H), jnp.float32),
        compiler_params=_SC_PARAMS,
        scratch_types=[
            pltpu.VMEM((C,), jnp.int32),
            pltpu.VMEM((C,), jnp.int32),
            pltpu.VMEM((C, _H), jnp.float32),
            pltpu.VMEM((C, _H), jnp.float32),
            pltpu.VMEM_SHARED((N, _H), jnp.float32),
            pltpu.SemaphoreType.DMA,
            pltpu.SemaphoreType.DMA,
        ],
    )
    def k(h_hbm, dst_hbm, out_hbm, idx0, idx1, rb0, rb1, shared, rsem, ssem):
        c = lax.axis_index("c")
        s = lax.axis_index("s")
        wid = s * NC + c
        base = wid * RW
        hf = wid // NS
        prow = base - hf * (E // 2)
        idxb = (idx0, idx1)
        rbb = (rb0, rb1)

        def zrow(i, carry):
            def zvec(j, carry2):
                rb1[i, pl.ds(j * 16, 16)] = jnp.zeros((16,), jnp.float32)
                return carry2
            return lax.fori_loop(0, _H // 16, zvec, carry)
        lax.fori_loop(0, C, zrow, 0)
        pltpu.sync_copy(rb1, shared.at[pl.ds(s * NPT, C)])
        pltpu.sync_copy(rb1.at[pl.ds(0, NPT - C)],
                        shared.at[pl.ds(s * NPT + C, NPT - C)])
        plsc.subcore_barrier()

        def ldidx(g, b):
            pltpu.sync_copy(dst_hbm.at[pl.ds(base + g * C, C)], idxb[b])

        def ldrows(g, b):
            pltpu.async_copy(h_hbm.at[pl.ds(prow + g * C, C),
                                      pl.ds(hf * _H, _H)], rbb[b], rsem)

        def scat(b):
            pltpu.async_copy(rbb[b], shared.at[idxb[b]], ssem, add=True)

        def drain(sem):
            pltpu.make_async_copy(h_hbm.at[pl.ds(0, C), pl.ds(0, _H)],
                                  rb0, sem).wait()

        ldidx(0, 0)
        ldrows(0, 0)

        def body(t, first, last):
            g = 2 * t
            drain(rsem)
            if not first:
                drain(ssem)
            ldidx(g + 1, 1)
            ldrows(g + 1, 1)
            scat(0)
            drain(rsem)
            drain(ssem)
            if not last:
                ldidx(g + 2, 0)
                ldrows(g + 2, 0)
            scat(1)

        body(0, True, False)

        def fbody(t, carry):
            body(t, False, False)
            return carry
        lax.fori_loop(1, n // 2 - 1, fbody, 0)

        body(n // 2 - 1, False, True)
        drain(ssem)
        plsc.subcore_barrier()

        pltpu.sync_copy(shared.at[pl.ds(s * NPT, NPT)],
                        out_hbm.at[c].at[pl.ds(s * NPT, NPT)])

    return k


def _mm_kernel(a_ref, w_ref, o_ref):
    o_ref[...] = jnp.dot(a_ref[...], w_ref[...],
                         preferred_element_type=jnp.float32)


def _tc_matmul(a, w, block_rows):
    M, K = a.shape
    _, Np = w.shape
    grid = M // block_rows
    return pl.pallas_call(
        _mm_kernel,
        grid=(grid,),
        in_specs=[
            pl.BlockSpec((block_rows, K), lambda i: (i, 0)),
            pl.BlockSpec((K, Np), lambda i: (0, 0)),
        ],
        out_specs=pl.BlockSpec((block_rows, Np), lambda i: (i, 0)),
        out_shape=jax.ShapeDtypeStruct((M, Np), jnp.float32),
    )(a, w)


def _h0_kernel(g_ref, ea_lo_ref, ea_hi_ref, w1e_ref, h0_ref):
    w1e = w1e_ref[...]
    el = jnp.dot(ea_lo_ref[...], w1e, preferred_element_type=jnp.float32)
    er = jnp.dot(ea_hi_ref[...], w1e, preferred_element_type=jnp.float32)
    h0_ref[:, :_H] = jax.nn.relu(g_ref[:, :_H] + el)
    h0_ref[:, _H:] = jax.nn.relu(g_ref[:, _H:] + er)


def _tc_h0(g0p, edge_attr, w1e, block_rows):
    grid = _E2 // block_rows
    nhalf = _E2 // block_rows
    EI = edge_attr.shape[1]
    pspec = pl.BlockSpec((block_rows, 128), lambda i: (i, 0))
    return pl.pallas_call(
        _h0_kernel,
        grid=(grid,),
        in_specs=[
            pspec,
            pl.BlockSpec((block_rows, EI), lambda i: (i, 0)),
            pl.BlockSpec((block_rows, EI), lambda i: (i + nhalf, 0)),
            pl.BlockSpec((EI, _H), lambda i: (0, 0)),
        ],
        out_specs=pspec,
        out_shape=jax.ShapeDtypeStruct((_E2, 128), jnp.float32),
    )(g0p, edge_attr, edge_attr, w1e)


def _combine_kernel(h0_ref, g1_ref, g2_ref, w_ref, h_ref):
    d = g1_ref[...] - g2_ref[...]
    w2 = w_ref[...]
    ml = jnp.dot(d[:, :_H], w2, preferred_element_type=jnp.float32)
    mr = jnp.dot(d[:, _H:], w2, preferred_element_type=jnp.float32)
    h_ref[:, :_H] = jax.nn.relu(h0_ref[:, :_H] + ml)
    h_ref[:, _H:] = jax.nn.relu(h0_ref[:, _H:] + mr)


def _tc_combine(h0p, g1p, g2p, w2, block_rows):
    grid = _E2 // block_rows
    pspec = pl.BlockSpec((block_rows, 128), lambda i: (i, 0))
    return pl.pallas_call(
        _combine_kernel,
        grid=(grid,),
        in_specs=[pspec, pspec, pspec,
                  pl.BlockSpec((_H, _H), lambda i: (0, 0))],
        out_specs=pspec,
        out_shape=jax.ShapeDtypeStruct((_E2, 128), jnp.float32),
    )(h0p, g1p, g2p, w2)


def _psum_kernel(p_ref, o_ref):
    o_ref[...] = p_ref[0] + p_ref[1]


def _tc_psum(p, block_rows):
    N = p.shape[1]
    grid = N // block_rows
    return pl.pallas_call(
        _psum_kernel,
        grid=(grid,),
        in_specs=[pl.BlockSpec((2, block_rows, _H), lambda i: (0, i, 0))],
        out_specs=pl.BlockSpec((block_rows, _H), lambda i: (i, 0)),
        out_shape=jax.ShapeDtypeStruct((N, _H), jnp.float32),
    )(p)


def _node_kernel(x_ref, p_ref, w3a_ref, w3b_ref, b3_ref, o_ref):
    v = p_ref[0] + p_ref[1]
    z = (jnp.dot(x_ref[...], w3a_ref[...], preferred_element_type=jnp.float32)
         + jnp.dot(v, w3b_ref[...], preferred_element_type=jnp.float32)
         + b3_ref[...])
    o_ref[...] = jax.nn.relu(z)


def _tc_node(x, p, w3a, w3b, b3, block_rows):
    N, K = x.shape
    grid = N // block_rows
    return pl.pallas_call(
        _node_kernel,
        grid=(grid,),
        in_specs=[
            pl.BlockSpec((block_rows, K), lambda i: (i, 0)),
            pl.BlockSpec((2, block_rows, _H), lambda i: (0, i, 0)),
            pl.BlockSpec((K, _H), lambda i: (0, 0)),
            pl.BlockSpec((_H, _H), lambda i: (0, 0)),
            pl.BlockSpec((1, _H), lambda i: (0, 0)),
        ],
        out_specs=pl.BlockSpec((block_rows, _H), lambda i: (i, 0)),
        out_shape=jax.ShapeDtypeStruct((N, _H), jnp.float32),
    )(x, p, w3a, w3b, b3)


def _tail_kernel(na_ref, batch_ref, wm1_ref, bm1_ref, wm2_ref, bm2_ref, o_ref):
    b = batch_ref[...]
    gids = lax.broadcasted_iota(jnp.int32, (_NUM_GRAPHS, b.shape[1]), 0)
    oh = (gids == b).astype(jnp.float32)
    sums = jnp.dot(oh, na_ref[...], preferred_element_type=jnp.float32,
                   precision=lax.Precision.HIGHEST)
    counts = jnp.sum(oh, axis=1, keepdims=True)
    pooled = sums / jnp.maximum(counts, 1.0)
    z1 = jax.nn.relu(
        jnp.dot(pooled, wm1_ref[...], preferred_element_type=jnp.float32)
        + bm1_ref[...])
    out = (jnp.dot(z1, wm2_ref[...], preferred_element_type=jnp.float32)
           + bm2_ref[...])
    o_ref[...] = out * (1.0 / jnp.sqrt(jnp.float32(1.0 + 1e-5)))


def _tc_tail(node_attr, batch2d, Wm1, bm1, Wm2, bm2):
    return pl.pallas_call(
        _tail_kernel,
        out_shape=jax.ShapeDtypeStruct((_NUM_GRAPHS, 1), jnp.float32),
    )(node_attr, batch2d, Wm1, bm1.reshape(1, _H), Wm2, bm2.reshape(1, 1))


_sc_cache = {}


def _get_sc_kernels():
    if not _sc_cache:
        _sc_cache["gn"] = _make_sc_gather(_N, _E, 800)
        _sc_cache["ge"] = _make_sc_gather(_E, _E, 800)
        _sc_cache["ss"] = _make_sc_segsum(_E, _N, 400)
    return _sc_cache["gn"], _sc_cache["ge"], _sc_cache["ss"]


def kernel(x, edge_attr, W1, W2, W3, b3, Wm1, bm1, Wm2, bm2,
           edge_index, revedge_index, batch, num_nodes):
    gather_n, gather_e, segsum = _get_sc_kernels()
    src = edge_index[0]
    dst = edge_index[1]
    B_E = 2000
    B_N = 2000

    W1x = W1[:x.shape[1]]
    W1e = W1[x.shape[1]:]
    W3a = W3[:x.shape[1]]
    W3b = W3[x.shape[1]:]

    rev2 = 2 * (revedge_index % _E2) + revedge_index // _E2

    xW1 = _tc_matmul(x, W1x, B_N)
    g0p = gather_n(xW1, src)
    h0p = _tc_h0(g0p, edge_attr, W1e, B_E)

    hp = h0p
    for it in range(_DEPTH - 1):
        p = segsum(hp, dst)
        m = _tc_psum(p, B_N)
        g1p = gather_n(m, src)
        g2p = gather_e(jnp.reshape(hp, (_E, _H)), rev2)
        hp = _tc_combine(h0p, g1p, g2p, W2, B_E)

    p = segsum(hp, dst)
    node_attr = _tc_node(x, p, W3a, W3b, b3.reshape(1, _H), B_N)
    out = _tc_tail(node_attr, batch.reshape(1, -1), Wm1, bm1, Wm2, bm2)
    return out


# speedup vs baseline: 6.4695x; 1.0669x over previous
import functools
import jax
import jax.numpy as jnp
from jax import lax
from jax.experimental import pallas as pl
from jax.experimental.pallas import tpu as pltpu
from jax.experimental.pallas import tpu_sc as plsc

NC = 2
NS = 16
NW = NC * NS

_E = 640000
_E2 = _E // 2
_N = 10000
_H = 64
_DEPTH = 3
_NUM_GRAPHS = 64

_SC_PARAMS = pltpu.CompilerParams(use_tc_tiling_on_sc=False)


def _make_sc_gather(T, E, C):
    RW = E // NW
    E2 = E // 2
    n = RW // C
    assert RW % C == 0 and n % 2 == 1
    npairs = (n - 1) // 2
    mesh = plsc.VectorSubcoreMesh(core_axis_name="c", subcore_axis_name="s",
                                  num_cores=NC, num_subcores=NS)

    @functools.partial(
        pl.kernel, mesh=mesh,
        out_type=jax.ShapeDtypeStruct((E2, 2 * _H), jnp.float32),
        compiler_params=_SC_PARAMS,
        scratch_types=[
            pltpu.VMEM((RW,), jnp.int32),
            pltpu.VMEM((C, _H), jnp.float32),
            pltpu.VMEM((C, _H), jnp.float32),
            pltpu.SemaphoreType.DMA,
            pltpu.SemaphoreType.DMA,
        ],
    )
    def k(table_hbm, idx_hbm, out_hbm, idx_all, rb0, rb1, gsem, wsem):
        wid = lax.axis_index("s") * NC + lax.axis_index("c")
        base = wid * RW
        hf = wid // NS
        prow = base - hf * E2
        pltpu.sync_copy(idx_hbm.at[pl.ds(base, RW)], idx_all)

        def idxs(g):
            return idx_all.at[pl.ds(g * C, C)]

        def outs(g):
            return out_hbm.at[pl.ds(prow + g * C, C), pl.ds(hf * _H, _H)]

        def drain(rb, sem):
            pltpu.make_async_copy(out_hbm.at[pl.ds(0, C), pl.ds(0, _H)],
                                  rb, sem).wait()

        pltpu.async_copy(table_hbm.at[idxs(0)], rb0, gsem)

        def body(t, first):
            g = 1 + 2 * t
            drain(rb0, gsem)
            if not first:
                drain(rb1, wsem)
            pltpu.async_copy(table_hbm.at[idxs(g)], rb1, gsem)
            pltpu.async_copy(rb0, outs(g - 1), wsem)
            drain(rb1, gsem)
            drain(rb0, wsem)
            pltpu.async_copy(table_hbm.at[idxs(g + 1)], rb0, gsem)
            pltpu.async_copy(rb1, outs(g), wsem)

        body(0, True)

        def fbody(t, carry):
            body(t, False)
            return carry
        lax.fori_loop(1, npairs, fbody, 0)

        drain(rb0, gsem)
        drain(rb1, wsem)
        pltpu.async_copy(rb0, outs(n - 1), wsem)
        drain(rb0, wsem)

    return k


def _make_sc_segsum(E, N, C):
    RW = E // NW
    n = RW // C
    assert RW % C == 0 and n % 2 == 0
    NPT = N // NS
    assert NPT > C and NPT < 2 * C
    mesh = plsc.VectorSubcoreMesh(core_axis_name="c", subcore_axis_name="s",
                                  num_cores=NC, num_subcores=NS)

    @functools.partial(
        pl.kernel, mesh=mesh,
        out_type=jax.ShapeDtypeStruct((NC, N, _H), jnp.float32),
        compiler_params=_SC_PARAMS,
        scratch_types=[
            pltpu.VMEM((C,), jnp.int32),
            pltpu.VMEM((C,), jnp.int32),
            pltpu.VMEM((C, _H), jnp.float32),
            pltpu.VMEM((C, _H), jnp.float32),
            pltpu.VMEM_SHARED((N, _H), jnp.float32),
            pltpu.SemaphoreType.DMA,
            pltpu.SemaphoreType.DMA,
        ],
    )
    def k(h_hbm, dst_hbm, out_hbm, idx0, idx1, rb0, rb1, shared, rsem, ssem):
        c = lax.axis_index("c")
        s = lax.axis_index("s")
        wid = s * NC + c
        base = wid * RW
        hf = wid // NS
        prow = base - hf * (E // 2)
        idxb = (idx0, idx1)
        rbb = (rb0, rb1)

        def zrow(i, carry):
            def zvec(j, carry2):
                rb1[i, pl.ds(j * 16, 16)] = jnp.zeros((16,), jnp.float32)
                return carry2
            return lax.fori_loop(0, _H // 16, zvec, carry)
        lax.fori_loop(0, C, zrow, 0)
        pltpu.sync_copy(rb1, shared.at[pl.ds(s * NPT, C)])
        pltpu.sync_copy(rb1.at[pl.ds(0, NPT - C)],
                        shared.at[pl.ds(s * NPT + C, NPT - C)])
        plsc.subcore_barrier()

        def ldidx(g, b):
            pltpu.sync_copy(dst_hbm.at[pl.ds(base + g * C, C)], idxb[b])

        def ldrows(g, b):
            pltpu.async_copy(h_hbm.at[pl.ds(prow + g * C, C),
                                      pl.ds(hf * _H, _H)], rbb[b], rsem)

        def scat(b):
            pltpu.async_copy(rbb[b], shared.at[idxb[b]], ssem, add=True)

        def drain(sem):
            pltpu.make_async_copy(h_hbm.at[pl.ds(0, C), pl.ds(0, _H)],
                                  rb0, sem).wait()

        ldidx(0, 0)
        ldrows(0, 0)

        def body(t, first, last):
            g = 2 * t
            drain(rsem)
            if not first:
                drain(ssem)
            ldidx(g + 1, 1)
            ldrows(g + 1, 1)
            scat(0)
            drain(rsem)
            drain(ssem)
            if not last:
                ldidx(g + 2, 0)
                ldrows(g + 2, 0)
            scat(1)

        body(0, True, False)

        def fbody(t, carry):
            body(t, False, False)
            return carry
        lax.fori_loop(1, n // 2 - 1, fbody, 0)

        body(n // 2 - 1, False, True)
        drain(ssem)
        plsc.subcore_barrier()

        pltpu.sync_copy(shared.at[pl.ds(s * NPT, NPT)],
                        out_hbm.at[c].at[pl.ds(s * NPT, NPT)])

    return k


def _mm_kernel(a_ref, w_ref, o_ref):
    o_ref[...] = jnp.dot(a_ref[...], w_ref[...],
                         preferred_element_type=jnp.float32)


def _tc_matmul(a, w, block_rows):
    M, K = a.shape
    _, Np = w.shape
    grid = M // block_rows
    return pl.pallas_call(
        _mm_kernel,
        grid=(grid,),
        in_specs=[
            pl.BlockSpec((block_rows, K), lambda i: (i, 0)),
            pl.BlockSpec((K, Np), lambda i: (0, 0)),
        ],
        out_specs=pl.BlockSpec((block_rows, Np), lambda i: (i, 0)),
        out_shape=jax.ShapeDtypeStruct((M, Np), jnp.float32),
    )(a, w)


def _h0_kernel(g_ref, ea_lo_ref, ea_hi_ref, w1e_ref, h0_ref):
    w1e = w1e_ref[...]
    el = jnp.dot(ea_lo_ref[...], w1e, preferred_element_type=jnp.float32)
    er = jnp.dot(ea_hi_ref[...], w1e, preferred_element_type=jnp.float32)
    h0_ref[:, :_H] = jax.nn.relu(g_ref[:, :_H] + el)
    h0_ref[:, _H:] = jax.nn.relu(g_ref[:, _H:] + er)


def _tc_h0(g0p, edge_attr, w1e, block_rows):
    grid = _E2 // block_rows
    nhalf = _E2 // block_rows
    EI = edge_attr.shape[1]
    pspec = pl.BlockSpec((block_rows, 128), lambda i: (i, 0))
    return pl.pallas_call(
        _h0_kernel,
        grid=(grid,),
        in_specs=[
            pspec,
            pl.BlockSpec((block_rows, EI), lambda i: (i, 0)),
            pl.BlockSpec((block_rows, EI), lambda i: (i + nhalf, 0)),
            pl.BlockSpec((EI, _H), lambda i: (0, 0)),
        ],
        out_specs=pspec,
        out_shape=jax.ShapeDtypeStruct((_E2, 128), jnp.float32),
    )(g0p, edge_attr, edge_attr, w1e)


def _combine_kernel(h0_ref, g1_ref, g2_ref, w_ref, h_ref):
    d = g1_ref[...] - g2_ref[...]
    w2 = w_ref[...]
    ml = jnp.dot(d[:, :_H], w2, preferred_element_type=jnp.float32)
    mr = jnp.dot(d[:, _H:], w2, preferred_element_type=jnp.float32)
    h_ref[:, :_H] = jax.nn.relu(h0_ref[:, :_H] + ml)
    h_ref[:, _H:] = jax.nn.relu(h0_ref[:, _H:] + mr)


def _tc_combine(h0p, g1p, g2p, w2, block_rows):
    grid = _E2 // block_rows
    pspec = pl.BlockSpec((block_rows, 128), lambda i: (i, 0))
    return pl.pallas_call(
        _combine_kernel,
        grid=(grid,),
        in_specs=[pspec, pspec, pspec,
                  pl.BlockSpec((_H, _H), lambda i: (0, 0))],
        out_specs=pspec,
        out_shape=jax.ShapeDtypeStruct((_E2, 128), jnp.float32),
    )(h0p, g1p, g2p, w2)


def _psum_kernel(p_ref, o_ref):
    o_ref[...] = p_ref[0] + p_ref[1]


def _tc_psum(p, block_rows):
    N = p.shape[1]
    grid = N // block_rows
    return pl.pallas_call(
        _psum_kernel,
        grid=(grid,),
        in_specs=[pl.BlockSpec((2, block_rows, _H), lambda i: (0, i, 0))],
        out_specs=pl.BlockSpec((block_rows, _H), lambda i: (i, 0)),
        out_shape=jax.ShapeDtypeStruct((N, _H), jnp.float32),
    )(p)


def _node_kernel(x_ref, p_ref, w3a_ref, w3b_ref, b3_ref, o_ref):
    v = p_ref[0] + p_ref[1]
    z = (jnp.dot(x_ref[...], w3a_ref[...], preferred_element_type=jnp.float32)
         + jnp.dot(v, w3b_ref[...], preferred_element_type=jnp.float32)
         + b3_ref[...])
    o_ref[...] = jax.nn.relu(z)


def _tc_node(x, p, w3a, w3b, b3, block_rows):
    N, K = x.shape
    grid = N // block_rows
    return pl.pallas_call(
        _node_kernel,
        grid=(grid,),
        in_specs=[
            pl.BlockSpec((block_rows, K), lambda i: (i, 0)),
            pl.BlockSpec((2, block_rows, _H), lambda i: (0, i, 0)),
            pl.BlockSpec((K, _H), lambda i: (0, 0)),
            pl.BlockSpec((_H, _H), lambda i: (0, 0)),
            pl.BlockSpec((1, _H), lambda i: (0, 0)),
        ],
        out_specs=pl.BlockSpec((block_rows, _H), lambda i: (i, 0)),
        out_shape=jax.ShapeDtypeStruct((N, _H), jnp.float32),
    )(x, p, w3a, w3b, b3)


def _tail_kernel(na_ref, batch_ref, wm1_ref, bm1_ref, wm2_ref, bm2_ref, o_ref):
    b = batch_ref[...]
    gids = lax.broadcasted_iota(jnp.int32, (_NUM_GRAPHS, b.shape[1]), 0)
    oh = (gids == b).astype(jnp.float32)
    sums = jnp.dot(oh, na_ref[...], preferred_element_type=jnp.float32,
                   precision=lax.Precision.HIGHEST)
    counts = jnp.sum(oh, axis=1, keepdims=True)
    pooled = sums / jnp.maximum(counts, 1.0)
    z1 = jax.nn.relu(
        jnp.dot(pooled, wm1_ref[...], preferred_element_type=jnp.float32)
        + bm1_ref[...])
    out = (jnp.dot(z1, wm2_ref[...], preferred_element_type=jnp.float32)
           + bm2_ref[...])
    o_ref[...] = out * (1.0 / jnp.sqrt(jnp.float32(1.0 + 1e-5)))


def _tc_tail(node_attr, batch2d, Wm1, bm1, Wm2, bm2):
    return pl.pallas_call(
        _tail_kernel,
        out_shape=jax.ShapeDtypeStruct((_NUM_GRAPHS, 1), jnp.float32),
    )(node_attr, batch2d, Wm1, bm1.reshape(1, _H), Wm2, bm2.reshape(1, 1))


_sc_cache = {}


def _get_sc_kernels():
    if not _sc_cache:
        _sc_cache["gn"] = _make_sc_gather(_N, _E, 800)
        _sc_cache["ge"] = _make_sc_gather(_E, _E, 800)
        _sc_cache["ss"] = _make_sc_segsum(_E, _N, 400)
    return _sc_cache["gn"], _sc_cache["ge"], _sc_cache["ss"]


def kernel(x, edge_attr, W1, W2, W3, b3, Wm1, bm1, Wm2, bm2,
           edge_index, revedge_index, batch, num_nodes):
    gather_n, gather_e, segsum = _get_sc_kernels()
    src = edge_index[0]
    dst = edge_index[1]
    B_E = 8000
    B_N = 2000

    W1x = W1[:x.shape[1]]
    W1e = W1[x.shape[1]:]
    W3a = W3[:x.shape[1]]
    W3b = W3[x.shape[1]:]

    rev2 = 2 * (revedge_index % _E2) + revedge_index // _E2

    xW1 = _tc_matmul(x, W1x, B_N)
    g0p = gather_n(xW1, src)
    h0p = _tc_h0(g0p, edge_attr, W1e, B_E)

    hp = h0p
    for it in range(_DEPTH - 1):
        p = segsum(hp, dst)
        m = _tc_psum(p, B_N)
        g1p = gather_n(m, src)
        g2p = gather_e(jnp.reshape(hp, (_E, _H)), rev2)
        hp = _tc_combine(h0p, g1p, g2p, W2, B_E)

    p = segsum(hp, dst)
    node_attr = _tc_node(x, p, W3a, W3b, b3.reshape(1, _H), B_N)
    out = _tc_tail(node_attr, batch.reshape(1, -1), Wm1, bm1, Wm2, bm2)
    return out


# speedup vs baseline: 6.4715x; 1.0003x over previous
import functools
import jax
import jax.numpy as jnp
from jax import lax
from jax.experimental import pallas as pl
from jax.experimental.pallas import tpu as pltpu
from jax.experimental.pallas import tpu_sc as plsc

NC = 2
NS = 16
NW = NC * NS

_E = 640000
_E2 = _E // 2
_N = 10000
_H = 64
_DEPTH = 3
_NUM_GRAPHS = 64

_SC_PARAMS = pltpu.CompilerParams(use_tc_tiling_on_sc=False)


def _make_sc_gather(T, E, C):
    RW = E // NW
    E2 = E // 2
    n = RW // C
    assert RW % C == 0 and n % 2 == 1
    npairs = (n - 1) // 2
    mesh = plsc.VectorSubcoreMesh(core_axis_name="c", subcore_axis_name="s",
                                  num_cores=NC, num_subcores=NS)

    @functools.partial(
        pl.kernel, mesh=mesh,
        out_type=jax.ShapeDtypeStruct((E2, 2 * _H), jnp.float32),
        compiler_params=_SC_PARAMS,
        scratch_types=[
            pltpu.VMEM((RW,), jnp.int32),
            pltpu.VMEM((C, _H), jnp.float32),
            pltpu.VMEM((C, _H), jnp.float32),
            pltpu.SemaphoreType.DMA,
            pltpu.SemaphoreType.DMA,
        ],
    )
    def k(table_hbm, idx_hbm, out_hbm, idx_all, rb0, rb1, gsem, wsem):
        wid = lax.axis_index("s") * NC + lax.axis_index("c")
        base = wid * RW
        hf = wid // NS
        prow = base - hf * E2
        pltpu.sync_copy(idx_hbm.at[pl.ds(base, RW)], idx_all)

        def idxs(g):
            return idx_all.at[pl.ds(g * C, C)]

        def outs(g):
            return out_hbm.at[pl.ds(prow + g * C, C), pl.ds(hf * _H, _H)]

        def drain(rb, sem):
            pltpu.make_async_copy(out_hbm.at[pl.ds(0, C), pl.ds(0, _H)],
                                  rb, sem).wait()

        pltpu.async_copy(table_hbm.at[idxs(0)], rb0, gsem)

        def body(t, first):
            g = 1 + 2 * t
            drain(rb0, gsem)
            if not first:
                drain(rb1, wsem)
            pltpu.async_copy(table_hbm.at[idxs(g)], rb1, gsem)
            pltpu.async_copy(rb0, outs(g - 1), wsem)
            drain(rb1, gsem)
            drain(rb0, wsem)
            pltpu.async_copy(table_hbm.at[idxs(g + 1)], rb0, gsem)
            pltpu.async_copy(rb1, outs(g), wsem)

        body(0, True)

        def fbody(t, carry):
            body(t, False)
            return carry
        lax.fori_loop(1, npairs, fbody, 0)

        drain(rb0, gsem)
        drain(rb1, wsem)
        pltpu.async_copy(rb0, outs(n - 1), wsem)
        drain(rb0, wsem)

    return k


def _make_sc_segsum(E, N, C):
    RW = E // NW
    n = RW // C
    assert RW % C == 0 and n % 2 == 0
    NPT = N // NS
    assert NPT > C and NPT < 2 * C
    mesh = plsc.VectorSubcoreMesh(core_axis_name="c", subcore_axis_name="s",
                                  num_cores=NC, num_subcores=NS)

    @functools.partial(
        pl.kernel, mesh=mesh,
        out_type=jax.ShapeDtypeStruct((NC, N, _H), jnp.float32),
        compiler_params=_SC_PARAMS,
        scratch_types=[
            pltpu.VMEM((C,), jnp.int32),
            pltpu.VMEM((C,), jnp.int32),
            pltpu.VMEM((C, _H), jnp.float32),
            pltpu.VMEM((C, _H), jnp.float32),
            pltpu.VMEM_SHARED((N, _H), jnp.float32),
            pltpu.SemaphoreType.DMA,
            pltpu.SemaphoreType.DMA,
        ],
    )
    def k(h_hbm, dst_hbm, out_hbm, idx0, idx1, rb0, rb1, shared, rsem, ssem):
        c = lax.axis_index("c")
        s = lax.axis_index("s")
        wid = s * NC + c
        base = wid * RW
        hf = wid // NS
        prow = base - hf * (E // 2)
        idxb = (idx0, idx1)
        rbb = (rb0, rb1)

        def zrow(i, carry):
            def zvec(j, carry2):
                rb1[i, pl.ds(j * 16, 16)] = jnp.zeros((16,), jnp.float32)
                return carry2
            return lax.fori_loop(0, _H // 16, zvec, carry)
        lax.fori_loop(0, C, zrow, 0)
        pltpu.sync_copy(rb1, shared.at[pl.ds(s * NPT, C)])
        pltpu.sync_copy(rb1.at[pl.ds(0, NPT - C)],
                        shared.at[pl.ds(s * NPT + C, NPT - C)])
        plsc.subcore_barrier()

        def ldidx(g, b):
            pltpu.sync_copy(dst_hbm.at[pl.ds(base + g * C, C)], idxb[b])

        def ldrows(g, b):
            pltpu.async_copy(h_hbm.at[pl.ds(prow + g * C, C),
                                      pl.ds(hf * _H, _H)], rbb[b], rsem)

        def scat(b):
            pltpu.async_copy(rbb[b], shared.at[idxb[b]], ssem, add=True)

        def drain(sem):
            pltpu.make_async_copy(h_hbm.at[pl.ds(0, C), pl.ds(0, _H)],
                                  rb0, sem).wait()

        ldidx(0, 0)
        ldrows(0, 0)

        def body(t, first, last):
            g = 2 * t
            drain(rsem)
            if not first:
                drain(ssem)
            ldidx(g + 1, 1)
            ldrows(g + 1, 1)
            scat(0)
            drain(rsem)
            drain(ssem)
            if not last:
                ldidx(g + 2, 0)
                ldrows(g + 2, 0)
            scat(1)

        body(0, True, False)

        def fbody(t, carry):
            body(t, False, False)
            return carry
        lax.fori_loop(1, n // 2 - 1, fbody, 0)

        body(n // 2 - 1, False, True)
        drain(ssem)
        plsc.subcore_barrier()

        pltpu.sync_copy(shared.at[pl.ds(s * NPT, NPT)],
                        out_hbm.at[c].at[pl.ds(s * NPT, NPT)])

    return k


def _mm_kernel(a_ref, w_ref, o_ref):
    o_ref[...] = jnp.dot(a_ref[...], w_ref[...],
                         preferred_element_type=jnp.float32)


def _tc_matmul(a, w, block_rows):
    M, K = a.shape
    _, Np = w.shape
    grid = M // block_rows
    return pl.pallas_call(
        _mm_kernel,
        grid=(grid,),
        in_specs=[
            pl.BlockSpec((block_rows, K), lambda i: (i, 0)),
            pl.BlockSpec((K, Np), lambda i: (0, 0)),
        ],
        out_specs=pl.BlockSpec((block_rows, Np), lambda i: (i, 0)),
        out_shape=jax.ShapeDtypeStruct((M, Np), jnp.float32),
    )(a, w)


def _h0_kernel(g_ref, ea_lo_ref, ea_hi_ref, w1e_ref, h0_ref):
    w1e = w1e_ref[...]
    el = jnp.dot(ea_lo_ref[...], w1e, preferred_element_type=jnp.float32)
    er = jnp.dot(ea_hi_ref[...], w1e, preferred_element_type=jnp.float32)
    h0_ref[:, :_H] = jax.nn.relu(g_ref[:, :_H] + el)
    h0_ref[:, _H:] = jax.nn.relu(g_ref[:, _H:] + er)


def _tc_h0(g0p, edge_attr, w1e, block_rows):
    grid = _E2 // block_rows
    nhalf = _E2 // block_rows
    EI = edge_attr.shape[1]
    pspec = pl.BlockSpec((block_rows, 128), lambda i: (i, 0))
    return pl.pallas_call(
        _h0_kernel,
        grid=(grid,),
        in_specs=[
            pspec,
            pl.BlockSpec((block_rows, EI), lambda i: (i, 0)),
            pl.BlockSpec((block_rows, EI), lambda i: (i + nhalf, 0)),
            pl.BlockSpec((EI, _H), lambda i: (0, 0)),
        ],
        out_specs=pspec,
        out_shape=jax.ShapeDtypeStruct((_E2, 128), jnp.float32),
    )(g0p, edge_attr, edge_attr, w1e)


def _combine_kernel(h0_ref, g1_ref, g2_ref, w_ref, h_ref):
    d = g1_ref[...] - g2_ref[...]
    w2 = w_ref[...]
    ml = jnp.dot(d[:, :_H], w2, preferred_element_type=jnp.float32)
    mr = jnp.dot(d[:, _H:], w2, preferred_element_type=jnp.float32)
    h_ref[:, :_H] = jax.nn.relu(h0_ref[:, :_H] + ml)
    h_ref[:, _H:] = jax.nn.relu(h0_ref[:, _H:] + mr)


def _tc_combine(h0p, g1p, g2p, w2, block_rows):
    grid = _E2 // block_rows
    pspec = pl.BlockSpec((block_rows, 128), lambda i: (i, 0))
    return pl.pallas_call(
        _combine_kernel,
        grid=(grid,),
        in_specs=[pspec, pspec, pspec,
                  pl.BlockSpec((_H, _H), lambda i: (0, 0))],
        out_specs=pspec,
        out_shape=jax.ShapeDtypeStruct((_E2, 128), jnp.float32),
    )(h0p, g1p, g2p, w2)


def _psum_kernel(p_ref, o_ref):
    o_ref[...] = p_ref[0] + p_ref[1]


def _tc_psum(p, block_rows):
    N = p.shape[1]
    grid = N // block_rows
    return pl.pallas_call(
        _psum_kernel,
        grid=(grid,),
        in_specs=[pl.BlockSpec((2, block_rows, _H), lambda i: (0, i, 0))],
        out_specs=pl.BlockSpec((block_rows, _H), lambda i: (i, 0)),
        out_shape=jax.ShapeDtypeStruct((N, _H), jnp.float32),
    )(p)


def _node_kernel(x_ref, p_ref, w3a_ref, w3b_ref, b3_ref, o_ref):
    v = p_ref[0] + p_ref[1]
    z = (jnp.dot(x_ref[...], w3a_ref[...], preferred_element_type=jnp.float32)
         + jnp.dot(v, w3b_ref[...], preferred_element_type=jnp.float32)
         + b3_ref[...])
    o_ref[...] = jax.nn.relu(z)


def _tc_node(x, p, w3a, w3b, b3, block_rows):
    N, K = x.shape
    grid = N // block_rows
    return pl.pallas_call(
        _node_kernel,
        grid=(grid,),
        in_specs=[
            pl.BlockSpec((block_rows, K), lambda i: (i, 0)),
            pl.BlockSpec((2, block_rows, _H), lambda i: (0, i, 0)),
            pl.BlockSpec((K, _H), lambda i: (0, 0)),
            pl.BlockSpec((_H, _H), lambda i: (0, 0)),
            pl.BlockSpec((1, _H), lambda i: (0, 0)),
        ],
        out_specs=pl.BlockSpec((block_rows, _H), lambda i: (i, 0)),
        out_shape=jax.ShapeDtypeStruct((N, _H), jnp.float32),
    )(x, p, w3a, w3b, b3)


def _tail_kernel(na_ref, batch_ref, wm1_ref, bm1_ref, wm2_ref, bm2_ref, o_ref):
    b = batch_ref[...]
    gids = lax.broadcasted_iota(jnp.int32, (_NUM_GRAPHS, b.shape[1]), 0)
    oh = (gids == b).astype(jnp.float32)
    sums = jnp.dot(oh, na_ref[...], preferred_element_type=jnp.float32,
                   precision=lax.Precision.HIGHEST)
    counts = jnp.sum(oh, axis=1, keepdims=True)
    pooled = sums / jnp.maximum(counts, 1.0)
    z1 = jax.nn.relu(
        jnp.dot(pooled, wm1_ref[...], preferred_element_type=jnp.float32)
        + bm1_ref[...])
    out = (jnp.dot(z1, wm2_ref[...], preferred_element_type=jnp.float32)
           + bm2_ref[...])
    o_ref[...] = out * (1.0 / jnp.sqrt(jnp.float32(1.0 + 1e-5)))


def _tc_tail(node_attr, batch2d, Wm1, bm1, Wm2, bm2):
    return pl.pallas_call(
        _tail_kernel,
        out_shape=jax.ShapeDtypeStruct((_NUM_GRAPHS, 1), jnp.float32),
    )(node_attr, batch2d, Wm1, bm1.reshape(1, _H), Wm2, bm2.reshape(1, 1))


_sc_cache = {}


def _get_sc_kernels():
    if not _sc_cache:
        _sc_cache["gn"] = _make_sc_gather(_N, _E, 800)
        _sc_cache["ge"] = _make_sc_gather(_E, _E, 800)
        _sc_cache["ss"] = _make_sc_segsum(_E, _N, 400)
    return _sc_cache["gn"], _sc_cache["ge"], _sc_cache["ss"]


def kernel(x, edge_attr, W1, W2, W3, b3, Wm1, bm1, Wm2, bm2,
           edge_index, revedge_index, batch, num_nodes):
    gather_n, gather_e, segsum = _get_sc_kernels()
    src = edge_index[0]
    dst = edge_index[1]
    B_E = 10000
    B_N = 10000

    W1x = W1[:x.shape[1]]
    W1e = W1[x.shape[1]:]
    W3a = W3[:x.shape[1]]
    W3b = W3[x.shape[1]:]

    rev2 = 2 * (revedge_index % _E2) + revedge_index // _E2

    xW1 = _tc_matmul(x, W1x, B_N)
    g0p = gather_n(xW1, src)
    h0p = _tc_h0(g0p, edge_attr, W1e, B_E)

    hp = h0p
    for it in range(_DEPTH - 1):
        p = segsum(hp, dst)
        m = _tc_psum(p, B_N)
        g1p = gather_n(m, src)
        g2p = gather_e(jnp.reshape(hp, (_E, _H)), rev2)
        hp = _tc_combine(h0p, g1p, g2p, W2, B_E)

    p = segsum(hp, dst)
    node_attr = _tc_node(x, p, W3a, W3b, b3.reshape(1, _H), B_N)
    out = _tc_tail(node_attr, batch.reshape(1, -1), Wm1, bm1, Wm2, bm2)
    return out


# speedup vs baseline: 6.9155x; 1.0686x over previous
import functools
import jax
import jax.numpy as jnp
from jax import lax
from jax.experimental import pallas as pl
from jax.experimental.pallas import tpu as pltpu
from jax.experimental.pallas import tpu_sc as plsc

NC = 2
NS = 16
NW = NC * NS

_E = 640000
_E2 = _E // 2
_N = 10000
_H = 64
_DEPTH = 3
_NUM_GRAPHS = 64

_SC_PARAMS = pltpu.CompilerParams(use_tc_tiling_on_sc=False)


def _make_sc_gather(T, E, C):
    RW = E // NW
    E2 = E // 2
    n = RW // C
    assert RW % C == 0 and n % 2 == 1
    npairs = (n - 1) // 2
    mesh = plsc.VectorSubcoreMesh(core_axis_name="c", subcore_axis_name="s",
                                  num_cores=NC, num_subcores=NS)

    @functools.partial(
        pl.kernel, mesh=mesh,
        out_type=jax.ShapeDtypeStruct((E2, 2 * _H), jnp.float32),
        compiler_params=_SC_PARAMS,
        scratch_types=[
            pltpu.VMEM((RW,), jnp.int32),
            pltpu.VMEM((C, _H), jnp.float32),
            pltpu.VMEM((C, _H), jnp.float32),
            pltpu.SemaphoreType.DMA,
            pltpu.SemaphoreType.DMA,
        ],
    )
    def k(table_hbm, idx_hbm, out_hbm, idx_all, rb0, rb1, gsem, wsem):
        wid = lax.axis_index("s") * NC + lax.axis_index("c")
        base = wid * RW
        hf = wid // NS
        prow = base - hf * E2
        pltpu.sync_copy(idx_hbm.at[pl.ds(base, RW)], idx_all)

        def idxs(g):
            return idx_all.at[pl.ds(g * C, C)]

        def outs(g):
            return out_hbm.at[pl.ds(prow + g * C, C), pl.ds(hf * _H, _H)]

        def drain(rb, sem):
            pltpu.make_async_copy(out_hbm.at[pl.ds(0, C), pl.ds(0, _H)],
                                  rb, sem).wait()

        pltpu.async_copy(table_hbm.at[idxs(0)], rb0, gsem)

        def body(t, first):
            g = 1 + 2 * t
            drain(rb0, gsem)
            if not first:
                drain(rb1, wsem)
            pltpu.async_copy(table_hbm.at[idxs(g)], rb1, gsem)
            pltpu.async_copy(rb0, outs(g - 1), wsem)
            drain(rb1, gsem)
            drain(rb0, wsem)
            pltpu.async_copy(table_hbm.at[idxs(g + 1)], rb0, gsem)
            pltpu.async_copy(rb1, outs(g), wsem)

        body(0, True)

        def fbody(t, carry):
            body(t, False)
            return carry
        lax.fori_loop(1, npairs, fbody, 0)

        drain(rb0, gsem)
        drain(rb1, wsem)
        pltpu.async_copy(rb0, outs(n - 1), wsem)
        drain(rb0, wsem)

    return k


def _make_sc_gather2sub(T, E, C):
    RW = E // NW
    E2 = E // 2
    n = RW // C
    assert RW % C == 0 and n % 2 == 0
    mesh = plsc.VectorSubcoreMesh(core_axis_name="c", subcore_axis_name="s",
                                  num_cores=NC, num_subcores=NS)

    @functools.partial(
        pl.kernel, mesh=mesh,
        out_type=jax.ShapeDtypeStruct((E2, 2 * _H), jnp.float32),
        compiler_params=_SC_PARAMS,
        scratch_types=[
            pltpu.VMEM((C,), jnp.int32),
            pltpu.VMEM((C,), jnp.int32),
            pltpu.VMEM((C,), jnp.int32),
            pltpu.VMEM((C,), jnp.int32),
            pltpu.VMEM((C, _H), jnp.float32),
            pltpu.VMEM((C, _H), jnp.float32),
            pltpu.VMEM((C, _H), jnp.float32),
            pltpu.VMEM((C, _H), jnp.float32),
            pltpu.SemaphoreType.DMA,
            pltpu.SemaphoreType.DMA,
        ],
    )
    def k(mt_hbm, ht_hbm, src_hbm, rev_hbm, out_hbm,
          src0, src1, rev0, rev1, ra0, ra1, rb0, rb1, gsem, wsem):
        wid = lax.axis_index("s") * NC + lax.axis_index("c")
        base = wid * RW
        hf = wid // NS
        prow = base - hf * E2
        srcb = (src0, src1)
        revb = (rev0, rev1)
        rab = (ra0, ra1)
        rbb = (rb0, rb1)

        def ldidx(g, b):
            pltpu.sync_copy(src_hbm.at[pl.ds(base + g * C, C)], srcb[b])
            pltpu.sync_copy(rev_hbm.at[pl.ds(base + g * C, C)], revb[b])

        def gath(b):
            pltpu.async_copy(mt_hbm.at[srcb[b]], rab[b], gsem)
            pltpu.async_copy(ht_hbm.at[revb[b]], rbb[b], gsem)

        def sub(b):
            ra, rb = rab[b], rbb[b]

            def srow(r, carry):
                for u in range(_H // 16):
                    sl = pl.ds(u * 16, 16)
                    ra[r, sl] = ra[r, sl] - rb[r, sl]
                return carry
            lax.fori_loop(0, C, srow, 0)

        def wout(g, b):
            pltpu.async_copy(
                rab[b], out_hbm.at[pl.ds(prow + g * C, C),
                                   pl.ds(hf * _H, _H)], wsem)

        def drain(sem, k2):
            for _ in range(k2):
                pltpu.make_async_copy(
                    out_hbm.at[pl.ds(0, C), pl.ds(0, _H)], ra0, sem).wait()

        ldidx(0, 0)
        gath(0)

        def body(t, first, last):
            g = 2 * t
            drain(gsem, 2)
            if not first:
                drain(wsem, 1)
            ldidx(g + 1, 1)
            gath(1)
            sub(0)
            wout(g, 0)
            drain(gsem, 2)
            drain(wsem, 1)
            if not last:
                ldidx(g + 2, 0)
                gath(0)
            sub(1)
            wout(g + 1, 1)

        body(0, True, False)

        def fbody(t, carry):
            body(t, False, False)
            return carry
        lax.fori_loop(1, n // 2 - 1, fbody, 0)

        body(n // 2 - 1, False, True)
        drain(wsem, 1)

    return k


def _make_sc_segsum(E, N, C):
    RW = E // NW
    n = RW // C
    assert RW % C == 0 and n % 2 == 0
    NPT = N // NS
    assert NPT > C and NPT < 2 * C
    mesh = plsc.VectorSubcoreMesh(core_axis_name="c", subcore_axis_name="s",
                                  num_cores=NC, num_subcores=NS)

    @functools.partial(
        pl.kernel, mesh=mesh,
        out_type=jax.ShapeDtypeStruct((NC, N, _H), jnp.float32),
        compiler_params=_SC_PARAMS,
        scratch_types=[
            pltpu.VMEM((C,), jnp.int32),
            pltpu.VMEM((C,), jnp.int32),
            pltpu.VMEM((C, _H), jnp.float32),
            pltpu.VMEM((C, _H), jnp.float32),
            pltpu.VMEM_SHARED((N, _H), jnp.float32),
            pltpu.SemaphoreType.DMA,
            pltpu.SemaphoreType.DMA,
        ],
    )
    def k(h_hbm, dst_hbm, out_hbm, idx0, idx1, rb0, rb1, shared, rsem, ssem):
        c = lax.axis_index("c")
        s = lax.axis_index("s")
        wid = s * NC + c
        base = wid * RW
        hf = wid // NS
        prow = base - hf * (E // 2)
        idxb = (idx0, idx1)
        rbb = (rb0, rb1)

        def zrow(i, carry):
            def zvec(j, carry2):
                rb1[i, pl.ds(j * 16, 16)] = jnp.zeros((16,), jnp.float32)
                return carry2
            return lax.fori_loop(0, _H // 16, zvec, carry)
        lax.fori_loop(0, C, zrow, 0)
        pltpu.sync_copy(rb1, shared.at[pl.ds(s * NPT, C)])
        pltpu.sync_copy(rb1.at[pl.ds(0, NPT - C)],
                        shared.at[pl.ds(s * NPT + C, NPT - C)])
        plsc.subcore_barrier()

        def ldidx(g, b):
            pltpu.sync_copy(dst_hbm.at[pl.ds(base + g * C, C)], idxb[b])

        def ldrows(g, b):
            pltpu.async_copy(h_hbm.at[pl.ds(prow + g * C, C),
                                      pl.ds(hf * _H, _H)], rbb[b], rsem)

        def scat(b):
            pltpu.async_copy(rbb[b], shared.at[idxb[b]], ssem, add=True)

        def drain(sem):
            pltpu.make_async_copy(h_hbm.at[pl.ds(0, C), pl.ds(0, _H)],
                                  rb0, sem).wait()

        ldidx(0, 0)
        ldrows(0, 0)

        def body(t, first, last):
            g = 2 * t
            drain(rsem)
            if not first:
                drain(ssem)
            ldidx(g + 1, 1)
            ldrows(g + 1, 1)
            scat(0)
            drain(rsem)
            drain(ssem)
            if not last:
                ldidx(g + 2, 0)
                ldrows(g + 2, 0)
            scat(1)

        body(0, True, False)

        def fbody(t, carry):
            body(t, False, False)
            return carry
        lax.fori_loop(1, n // 2 - 1, fbody, 0)

        body(n // 2 - 1, False, True)
        drain(ssem)
        plsc.subcore_barrier()

        pltpu.sync_copy(shared.at[pl.ds(s * NPT, NPT)],
                        out_hbm.at[c].at[pl.ds(s * NPT, NPT)])

    return k


def _mm_kernel(a_ref, w_ref, o_ref):
    o_ref[...] = jnp.dot(a_ref[...], w_ref[...],
                         preferred_element_type=jnp.float32)


def _tc_matmul(a, w, block_rows):
    M, K = a.shape
    _, Np = w.shape
    grid = M // block_rows
    return pl.pallas_call(
        _mm_kernel,
        grid=(grid,),
        in_specs=[
            pl.BlockSpec((block_rows, K), lambda i: (i, 0)),
            pl.BlockSpec((K, Np), lambda i: (0, 0)),
        ],
        out_specs=pl.BlockSpec((block_rows, Np), lambda i: (i, 0)),
        out_shape=jax.ShapeDtypeStruct((M, Np), jnp.float32),
    )(a, w)


def _h0_kernel(g_ref, ea_lo_ref, ea_hi_ref, w1e_ref, h0_ref):
    w1e = w1e_ref[...]
    el = jnp.dot(ea_lo_ref[...], w1e, preferred_element_type=jnp.float32)
    er = jnp.dot(ea_hi_ref[...], w1e, preferred_element_type=jnp.float32)
    h0_ref[:, :_H] = jax.nn.relu(g_ref[:, :_H] + el)
    h0_ref[:, _H:] = jax.nn.relu(g_ref[:, _H:] + er)


def _tc_h0(g0p, edge_attr, w1e, block_rows):
    grid = _E2 // block_rows
    nhalf = _E2 // block_rows
    EI = edge_attr.shape[1]
    pspec = pl.BlockSpec((block_rows, 128), lambda i: (i, 0))
    return pl.pallas_call(
        _h0_kernel,
        grid=(grid,),
        in_specs=[
            pspec,
            pl.BlockSpec((block_rows, EI), lambda i: (i, 0)),
            pl.BlockSpec((block_rows, EI), lambda i: (i + nhalf, 0)),
            pl.BlockSpec((EI, _H), lambda i: (0, 0)),
        ],
        out_specs=pspec,
        out_shape=jax.ShapeDtypeStruct((_E2, 128), jnp.float32),
    )(g0p, edge_attr, edge_attr, w1e)


def _combine_kernel(h0_ref, d_ref, w_ref, h_ref):
    d = d_ref[...]
    w2 = w_ref[...]
    ml = jnp.dot(d[:, :_H], w2, preferred_element_type=jnp.float32)
    mr = jnp.dot(d[:, _H:], w2, preferred_element_type=jnp.float32)
    h_ref[:, :_H] = jax.nn.relu(h0_ref[:, :_H] + ml)
    h_ref[:, _H:] = jax.nn.relu(h0_ref[:, _H:] + mr)


def _tc_combine(h0p, dp, w2, block_rows):
    grid = _E2 // block_rows
    pspec = pl.BlockSpec((block_rows, 128), lambda i: (i, 0))
    return pl.pallas_call(
        _combine_kernel,
        grid=(grid,),
        in_specs=[pspec, pspec,
                  pl.BlockSpec((_H, _H), lambda i: (0, 0))],
        out_specs=pspec,
        out_shape=jax.ShapeDtypeStruct((_E2, 128), jnp.float32),
    )(h0p, dp, w2)


def _psum_kernel(p_ref, o_ref):
    o_ref[...] = p_ref[0] + p_ref[1]


def _tc_psum(p, block_rows):
    N = p.shape[1]
    grid = N // block_rows
    return pl.pallas_call(
        _psum_kernel,
        grid=(grid,),
        in_specs=[pl.BlockSpec((2, block_rows, _H), lambda i: (0, i, 0))],
        out_specs=pl.BlockSpec((block_rows, _H), lambda i: (i, 0)),
        out_shape=jax.ShapeDtypeStruct((N, _H), jnp.float32),
    )(p)


def _node_kernel(x_ref, p_ref, w3a_ref, w3b_ref, b3_ref, o_ref):
    v = p_ref[0] + p_ref[1]
    z = (jnp.dot(x_ref[...], w3a_ref[...], preferred_element_type=jnp.float32)
         + jnp.dot(v, w3b_ref[...], preferred_element_type=jnp.float32)
         + b3_ref[...])
    o_ref[...] = jax.nn.relu(z)


def _tc_node(x, p, w3a, w3b, b3, block_rows):
    N, K = x.shape
    grid = N // block_rows
    return pl.pallas_call(
        _node_kernel,
        grid=(grid,),
        in_specs=[
            pl.BlockSpec((block_rows, K), lambda i: (i, 0)),
            pl.BlockSpec((2, block_rows, _H), lambda i: (0, i, 0)),
            pl.BlockSpec((K, _H), lambda i: (0, 0)),
            pl.BlockSpec((_H, _H), lambda i: (0, 0)),
            pl.BlockSpec((1, _H), lambda i: (0, 0)),
        ],
        out_specs=pl.BlockSpec((block_rows, _H), lambda i: (i, 0)),
        out_shape=jax.ShapeDtypeStruct((N, _H), jnp.float32),
    )(x, p, w3a, w3b, b3)


def _tail_kernel(na_ref, batch_ref, wm1_ref, bm1_ref, wm2_ref, bm2_ref, o_ref):
    b = batch_ref[...]
    gids = lax.broadcasted_iota(jnp.int32, (_NUM_GRAPHS, b.shape[1]), 0)
    oh = (gids == b).astype(jnp.float32)
    sums = jnp.dot(oh, na_ref[...], preferred_element_type=jnp.float32,
                   precision=lax.Precision.HIGHEST)
    counts = jnp.sum(oh, axis=1, keepdims=True)
    pooled = sums / jnp.maximum(counts, 1.0)
    z1 = jax.nn.relu(
        jnp.dot(pooled, wm1_ref[...], preferred_element_type=jnp.float32)
        + bm1_ref[...])
    out = (jnp.dot(z1, wm2_ref[...], preferred_element_type=jnp.float32)
           + bm2_ref[...])
    o_ref[...] = out * (1.0 / jnp.sqrt(jnp.float32(1.0 + 1e-5)))


def _tc_tail(node_attr, batch2d, Wm1, bm1, Wm2, bm2):
    return pl.pallas_call(
        _tail_kernel,
        out_shape=jax.ShapeDtypeStruct((_NUM_GRAPHS, 1), jnp.float32),
    )(node_attr, batch2d, Wm1, bm1.reshape(1, _H), Wm2, bm2.reshape(1, 1))


_sc_cache = {}


def _get_sc_kernels():
    if not _sc_cache:
        _sc_cache["gn"] = _make_sc_gather(_N, _E, 800)
        _sc_cache["g2s"] = _make_sc_gather2sub(_N, _E, 400)
        _sc_cache["ss"] = _make_sc_segsum(_E, _N, 400)
    return _sc_cache["gn"], _sc_cache["g2s"], _sc_cache["ss"]


def kernel(x, edge_attr, W1, W2, W3, b3, Wm1, bm1, Wm2, bm2,
           edge_index, revedge_index, batch, num_nodes):
    gather_n, gather2sub, segsum = _get_sc_kernels()
    src = edge_index[0]
    dst = edge_index[1]
    B_E = 10000
    B_N = 10000

    W1x = W1[:x.shape[1]]
    W1e = W1[x.shape[1]:]
    W3a = W3[:x.shape[1]]
    W3b = W3[x.shape[1]:]

    rev2 = 2 * (revedge_index % _E2) + revedge_index // _E2

    xW1 = _tc_matmul(x, W1x, B_N)
    g0p = gather_n(xW1, src)
    h0p = _tc_h0(g0p, edge_attr, W1e, B_E)

    hp = h0p
    for it in range(_DEPTH - 1):
        p = segsum(hp, dst)
        m = _tc_psum(p, B_N)
        dp = gather2sub(m, jnp.reshape(hp, (_E, _H)), src, rev2)
        hp = _tc_combine(h0p, dp, W2, B_E)

    p = segsum(hp, dst)
    node_attr = _tc_node(x, p, W3a, W3b, b3.reshape(1, _H), B_N)
    out = _tc_tail(node_attr, batch.reshape(1, -1), Wm1, bm1, Wm2, bm2)
    return out
